# Initial kernel scaffold; baseline (speedup 1.0000x reference)
import jax
import jax.numpy as jnp
from jax.experimental import pallas as pl


def kernel(x_h, adj, edge_index, pos_feat, run_base, W_pos, b_pos, W_pe, b_pe, W_init, b_init, W_gat, att_src, att_dst, b_gat, W_last, b_last):
    raise NotImplementedError("write your pallas kernel here")


# speedup vs baseline: 9.2708x; 9.2708x over previous
import functools

import jax
import jax.numpy as jnp
from jax import lax
from jax.experimental import pallas as pl
from jax.experimental.pallas import tpu as pltpu
from jax.experimental.pallas import tpu_sc as plsc

N = 10000
E = 160000
D_IN = 256
H = 256
HH = 128
PE_DIM = 16
C = 40
L_GCN = 4
L_PE = 2
ALPHA = 0.1
NEG_SLOPE = 0.2

NP_ = 10240
EG = 196608
RG = EG // 128
BLK = 512
NSC = 2
NSUB = 16
RPN = NP_ // NSUB

def _mesh():
  return plsc.VectorSubcoreMesh(
      core_axis_name="c", subcore_axis_name="s",
      num_cores=NSC, num_subcores=NSUB)

_f32 = jnp.float32
_i32 = jnp.int32


def _mm_split(xs, Ws, b, relu_in=False):
  nx = len(xs)

  def body(*refs):
    x_refs = refs[:nx]
    w_refs = refs[nx:2 * nx]
    b_ref = refs[2 * nx]
    o_ref = refs[2 * nx + 1]
    acc = None
    for xr, wr in zip(x_refs, w_refs):
      xv = xr[...]
      if relu_in:
        xv = jnp.maximum(xv, 0.0)
      p = jnp.dot(xv, wr[...], preferred_element_type=_f32)
      acc = p if acc is None else acc + p
    acc = acc + b_ref[...]
    o_ref[0] = acc[:, :HH]
    o_ref[1] = acc[:, HH:]

  in_specs = []
  for x in xs:
    k = x.shape[1]
    in_specs.append(pl.BlockSpec((BLK, k), lambda i: (i, 0)))
  for w in Ws:
    in_specs.append(pl.BlockSpec(w.shape, lambda i: (0, 0)))
  in_specs.append(pl.BlockSpec((1, H), lambda i: (0, 0)))
  return pl.pallas_call(
      body,
      grid=(NP_ // BLK,),
      in_specs=in_specs,
      out_specs=pl.BlockSpec((2, BLK, HH), lambda i: (0, i, 0)),
      out_shape=jax.ShapeDtypeStruct((2, NP_, HH), _f32),
  )(*xs, *Ws, b.reshape(1, H))


def _gat_pre(x, W, a_src, a_dst):
  def body(x0_ref, x1_ref, w0_ref, w1_ref, as_ref, ad_ref, xw_ref, asd_ref):
    xw = (jnp.dot(x0_ref[...], w0_ref[...], preferred_element_type=_f32) +
          jnp.dot(x1_ref[...], w1_ref[...], preferred_element_type=_f32))
    xw_ref[0] = xw[:, :HH]
    xw_ref[1] = xw[:, HH:]
    asd_ref[0] = jnp.sum(xw * as_ref[...], axis=1)
    asd_ref[1] = jnp.sum(xw * ad_ref[...], axis=1)

  return pl.pallas_call(
      body,
      grid=(NP_ // BLK,),
      in_specs=[
          pl.BlockSpec((BLK, HH), lambda i: (i, 0)),
          pl.BlockSpec((BLK, HH), lambda i: (i, 0)),
          pl.BlockSpec((HH, H), lambda i: (0, 0)),
          pl.BlockSpec((HH, H), lambda i: (0, 0)),
          pl.BlockSpec((1, H), lambda i: (0, 0)),
          pl.BlockSpec((1, H), lambda i: (0, 0)),
      ],
      out_specs=[
          pl.BlockSpec((2, BLK, HH), lambda i: (0, i, 0)),
          pl.BlockSpec((2, BLK), lambda i: (0, i)),
      ],
      out_shape=[
          jax.ShapeDtypeStruct((2, NP_, HH), _f32),
          jax.ShapeDtypeStruct((2, NP_), _f32),
      ],
  )(x[0], x[1], W[:HH], W[HH:], a_src.reshape(1, H), a_dst.reshape(1, H))


def _gat_epilogue(un, den, b, pos, rbf, last):
  def body(u_ref, d_ref, b_ref, p_ref, r_ref, o_ref):
    den = d_ref[0] + d_ref[1] + 1e-16
    for h in range(2):
      xh = u_ref[h] / den[:, None] + b_ref[0, h * HH:(h + 1) * HH]
      if not last:
        rb = r_ref[0, 0]
        mix = xh * (1.0 - ALPHA) + p_ref[h] * ALPHA
        xh = jnp.maximum(rb * mix + (1.0 - rb) * xh, 0.0)
      o_ref[h] = xh

  return pl.pallas_call(
      body,
      grid=(NP_ // BLK,),
      in_specs=[
          pl.BlockSpec((2, BLK, HH), lambda i: (0, i, 0)),
          pl.BlockSpec((2, BLK), lambda i: (0, i)),
          pl.BlockSpec((1, H), lambda i: (0, 0)),
          pl.BlockSpec((2, BLK, HH), lambda i: (0, i, 0)),
          pl.BlockSpec((1, 1), lambda i: (0, 0)),
      ],
      out_specs=pl.BlockSpec((2, BLK, HH), lambda i: (0, i, 0)),
      out_shape=jax.ShapeDtypeStruct((2, NP_, HH), _f32),
  )(un, den, b.reshape(1, H), pos, rbf)


def _pe_finish(h):
  def body(h_ref, o_ref):
    o_ref[...] = jnp.tanh(jnp.maximum(h_ref[...], 0.0))

  return pl.pallas_call(
      body,
      grid=(NP_ // BLK,),
      in_specs=[pl.BlockSpec((2, BLK, HH), lambda i: (0, i, 0))],
      out_specs=pl.BlockSpec((2, BLK, HH), lambda i: (0, i, 0)),
      out_shape=jax.ShapeDtypeStruct((2, NP_, HH), _f32),
  )(h)


def _final(x, W, b):
  def body(x0_ref, x1_ref, w0_ref, w1_ref, b_ref, e_ref, l_ref):
    e = (jnp.dot(x0_ref[...], w0_ref[...], preferred_element_type=_f32) +
         jnp.dot(x1_ref[...], w1_ref[...], preferred_element_type=_f32) +
         b_ref[...])
    m = jnp.max(e, axis=1, keepdims=True)
    z = e - m
    lse = jnp.log(jnp.sum(jnp.exp(z), axis=1, keepdims=True))
    e_ref[...] = e
    l_ref[...] = z - lse

  return pl.pallas_call(
      body,
      grid=(NP_ // BLK,),
      in_specs=[
          pl.BlockSpec((BLK, HH), lambda i: (i, 0)),
          pl.BlockSpec((BLK, HH), lambda i: (i, 0)),
          pl.BlockSpec((HH, C), lambda i: (0, 0)),
          pl.BlockSpec((HH, C), lambda i: (0, 0)),
          pl.BlockSpec((1, C), lambda i: (0, 0)),
      ],
      out_specs=[
          pl.BlockSpec((BLK, C), lambda i: (i, 0)),
          pl.BlockSpec((BLK, C), lambda i: (i, 0)),
      ],
      out_shape=[
          jax.ShapeDtypeStruct((NP_, C), _f32),
          jax.ShapeDtypeStruct((NP_, C), _f32),
      ],
  )(x[0], x[1], W[:HH], W[HH:], b.reshape(1, C))


_RW = RG // (NSC * NSUB)


def _build_edge_logits():
  return functools.partial(
      pl.kernel,
      out_type=(
          jax.ShapeDtypeStruct((RG, 128), _f32),
          jax.ShapeDtypeStruct((NSC * NP_,), _f32),
      ),
      mesh=_mesh(),
      compiler_params=pltpu.CompilerParams(needs_layout_passes=False),
      scratch_types=[
          pltpu.VMEM((NP_,), _f32),
          pltpu.VMEM((NP_,), _f32),
          pltpu.VMEM((_RW, 128), _i32),
          pltpu.VMEM((_RW, 128), _i32),
          pltpu.VMEM((_RW, 128), _f32),
          pltpu.VMEM((NP_,), _f32),
          pltpu.VMEM((RPN,), _f32),
          pltpu.VMEM((RPN,), _f32),
          pltpu.VMEM_SHARED((NSUB, NP_), _f32),
      ],
  )(_edge_logits_body)


def _edge_logits_body(asd_hbm, src_hbm, dst_hbm, ex_hbm, den_hbm,
                      as_v, ad_v, src_v, dst_v, ex_v, dp_v, acc_v, red_v,
                      shared):
  c = lax.axis_index("c")
  s = lax.axis_index("s")
  wid = s * NSC + c
  pltpu.sync_copy(asd_hbm.at[pl.ds(0, NP_)], as_v)
  pltpu.sync_copy(asd_hbm.at[pl.ds(NP_, NP_)], ad_v)
  pltpu.sync_copy(src_hbm.at[pl.ds(wid * _RW, _RW)], src_v)
  pltpu.sync_copy(dst_hbm.at[pl.ds(wid * _RW, _RW)], dst_v)

  def zero16(i, _):
    dp_v[pl.ds(i * 16, 16)] = jnp.zeros((16,), _f32)
    return 0
  lax.fori_loop(0, NP_ // 16, zero16, 0)

  def row(r, _):
    for k in range(8):
      sl = pl.ds(k * 16, 16)
      si = src_v[r, sl]
      di = dst_v[r, sl]
      ev = plsc.load_gather(as_v, [si]) + plsc.load_gather(ad_v, [di])
      ev = jnp.where(ev >= 0.0, ev, ev * NEG_SLOPE)
      ex = jnp.exp(ev)
      ex_v[r, sl] = ex
      plsc.addupdate_scatter(dp_v, [di], ex)
    return 0
  lax.fori_loop(0, _RW, row, 0)

  pltpu.sync_copy(ex_v, ex_hbm.at[pl.ds(wid * _RW, _RW)])
  pltpu.sync_copy(dp_v, shared.at[s])
  plsc.subcore_barrier()

  def zacc(i, _):
    acc_v[pl.ds(i * 16, 16)] = jnp.zeros((16,), _f32)
    return 0
  lax.fori_loop(0, RPN // 16, zacc, 0)
  for i in range(NSUB):
    pltpu.sync_copy(shared.at[i, pl.ds(s * RPN, RPN)], red_v)

    def addv(j, _):
      sl = pl.ds(j * 16, 16)
      acc_v[sl] = acc_v[sl] + red_v[sl]
      return 0
    lax.fori_loop(0, RPN // 16, addv, 0)
  pltpu.sync_copy(acc_v, den_hbm.at[pl.ds(c * NP_ + s * RPN, RPN)])


CH = 8


def _make_wscatter(R):
  rpw = R // NSUB

  def wscatter(tbl_hbm, w_hbm, src_hbm, dst_hbm, out_hbm,
               src_v, dst_v, w_v, rows_v, acc, sem):
    c = lax.axis_index("c")
    s = lax.axis_index("s")

    def zrow(r, _):
      for k in range(8):
        rows_v[r, pl.ds(k * 16, 16)] = jnp.zeros((16,), _f32)
      return 0
    lax.fori_loop(0, 128, zrow, 0)
    for i in range(RPN // 128):
      pltpu.sync_copy(rows_v, acc.at[pl.ds(s * RPN + i * 128, 128)])

    plsc.subcore_barrier()

    def chunk(ci, _):
      base = s * rpw + ci * CH
      pltpu.sync_copy(src_hbm.at[pl.ds(base, CH)], src_v)
      pltpu.sync_copy(dst_hbm.at[pl.ds(base, CH)], dst_v)
      pltpu.sync_copy(w_hbm.at[pl.ds(base * 128, CH * 128)], w_v)

      def adj(r, _):
        for k in range(8):
          sl = pl.ds(k * 16, 16)
          src_v[r, sl] = src_v[r, sl] + c * NP_
        return 0
      lax.fori_loop(0, CH, adj, 0)

      def step(j, _):
        pltpu.async_copy(tbl_hbm.at[src_v.at[j]], rows_v, sem).wait()

        def scale(e, _):
          wb = plsc.load_gather(w_v, [jnp.full((16,), j * 128 + e, _i32)])
          for k in range(8):
            sl = pl.ds(k * 16, 16)
            rows_v[e, sl] = rows_v[e, sl] * wb
          return 0
        lax.fori_loop(0, 128, scale, 0)
        pltpu.sync_copy(rows_v, acc.at[dst_v.at[j]], add=True)
        return 0
      lax.fori_loop(0, CH, step, 0)
      return 0
    lax.fori_loop(0, rpw // CH, chunk, 0)

    plsc.subcore_barrier()
    pltpu.sync_copy(acc.at[pl.ds(s * RPN, RPN)],
                    out_hbm.at[pl.ds(c * NP_ + s * RPN, RPN)])

  return functools.partial(
      pl.kernel,
      out_type=jax.ShapeDtypeStruct((NSC * NP_, HH), _f32),
      mesh=_mesh(),
      compiler_params=pltpu.CompilerParams(needs_layout_passes=False),
      scratch_types=[
          pltpu.VMEM((CH, 128), _i32),
          pltpu.VMEM((CH, 128), _i32),
          pltpu.VMEM((CH * 128,), _f32),
          pltpu.VMEM((128, HH), _f32),
          pltpu.VMEM_SHARED((NP_, HH), _f32),
          pltpu.SemaphoreType.DMA,
      ],
  )(wscatter)


def kernel(x_h, adj, edge_index, pos_feat, run_base, W_pos, b_pos, W_pe, b_pe,
           W_init, b_init, W_gat, att_src, att_dst, b_gat, W_last, b_last):
  ei = edge_index.astype(_i32)
  loops = jnp.arange(N, dtype=_i32)
  padg = N + jnp.arange(EG - E - N, dtype=_i32) % (NP_ - N)
  src_g = jnp.concatenate([ei[0], loops, padg]).reshape(RG, 128)
  dst_g = jnp.concatenate([ei[1], loops, padg]).reshape(RG, 128)
  pads = N + jnp.arange(EG - E, dtype=_i32) % (NP_ - N)
  src_s = jnp.concatenate([ei[0], pads]).reshape(RG, 128)
  dst_s = jnp.concatenate([ei[1], pads]).reshape(RG, 128)
  w_s = jnp.concatenate([adj, jnp.zeros((EG - E,), _f32)])

  x_h_p = jnp.pad(x_h, ((0, NP_ - N), (0, 0)))
  pos_p = jnp.pad(pos_feat, ((0, NP_ - N), (0, 0)))
  rbf = jnp.asarray(jnp.asarray(run_base) == 0, _f32).reshape(1, 1)

  _edge_logits = _build_edge_logits()
  _wscatter = _make_wscatter(RG)

  h = _mm_split([pos_p], [W_pos], b_pos)
  for i in range(L_PE):
    h = _mm_split([h[0], h[1]], [W_pe[i][:HH], W_pe[i][HH:]], b_pe[i],
                  relu_in=(i > 0))
    h = _wscatter(h.reshape(NSC * NP_, HH), w_s, src_s, dst_s)
    h = h.reshape(2, NP_, HH)
  pos_split = _pe_finish(h)

  x = _mm_split([x_h_p], [W_init], b_init)
  for i in range(L_GCN):
    xw, a_sd = _gat_pre(x, W_gat[i], att_src[i], att_dst[i])
    ex, den = _edge_logits(a_sd.reshape(NSC * NP_), src_g, dst_g)
    un = _wscatter(xw.reshape(NSC * NP_, HH), ex.reshape(EG), src_g, dst_g)
    x = _gat_epilogue(un.reshape(2, NP_, HH), den.reshape(2, NP_), b_gat[i],
                      pos_split, rbf, last=(i == L_GCN - 1))

  emb, logp = _final(x, W_last, b_last)
  return emb[:N], logp[:N]


# speedup vs baseline: 13.1063x; 1.4137x over previous
import functools

import jax
import jax.numpy as jnp
from jax import lax
from jax.experimental import pallas as pl
from jax.experimental.pallas import tpu as pltpu
from jax.experimental.pallas import tpu_sc as plsc

N = 10000
E = 160000
D_IN = 256
H = 256
HH = 128
PE_DIM = 16
C = 40
L_GCN = 4
L_PE = 2
ALPHA = 0.1
NEG_SLOPE = 0.2

NP_ = 10240
EG = 180224
RG = EG // 128
BLK = 512
NSC = 2
NSUB = 16
RPN = NP_ // NSUB

def _mesh():
  return plsc.VectorSubcoreMesh(
      core_axis_name="c", subcore_axis_name="s",
      num_cores=NSC, num_subcores=NSUB)

_f32 = jnp.float32
_i32 = jnp.int32


def _mm_split(xs, Ws, b, relu_in=False):
  nx = len(xs)

  def body(*refs):
    x_refs = refs[:nx]
    w_refs = refs[nx:2 * nx]
    b_ref = refs[2 * nx]
    o_ref = refs[2 * nx + 1]
    acc = None
    for xr, wr in zip(x_refs, w_refs):
      xv = xr[...]
      if relu_in:
        xv = jnp.maximum(xv, 0.0)
      p = jnp.dot(xv, wr[...], preferred_element_type=_f32)
      acc = p if acc is None else acc + p
    acc = acc + b_ref[...]
    o_ref[0] = acc[:, :HH]
    o_ref[1] = acc[:, HH:]

  in_specs = []
  for x in xs:
    k = x.shape[1]
    in_specs.append(pl.BlockSpec((BLK, k), lambda i: (i, 0)))
  for w in Ws:
    in_specs.append(pl.BlockSpec(w.shape, lambda i: (0, 0)))
  in_specs.append(pl.BlockSpec((1, H), lambda i: (0, 0)))
  return pl.pallas_call(
      body,
      grid=(NP_ // BLK,),
      in_specs=in_specs,
      out_specs=pl.BlockSpec((2, BLK, HH), lambda i: (0, i, 0)),
      out_shape=jax.ShapeDtypeStruct((2, NP_, HH), _f32),
  )(*xs, *Ws, b.reshape(1, H))


def _gat_pre(x, W, a_src, a_dst):
  def body(x0_ref, x1_ref, w0_ref, w1_ref, as_ref, ad_ref, xw_ref, asd_ref):
    xw = (jnp.dot(x0_ref[...], w0_ref[...], preferred_element_type=_f32) +
          jnp.dot(x1_ref[...], w1_ref[...], preferred_element_type=_f32))
    xw_ref[0] = xw[:, :HH]
    xw_ref[1] = xw[:, HH:]
    asd_ref[0] = jnp.sum(xw * as_ref[...], axis=1)
    asd_ref[1] = jnp.sum(xw * ad_ref[...], axis=1)

  return pl.pallas_call(
      body,
      grid=(NP_ // BLK,),
      in_specs=[
          pl.BlockSpec((BLK, HH), lambda i: (i, 0)),
          pl.BlockSpec((BLK, HH), lambda i: (i, 0)),
          pl.BlockSpec((HH, H), lambda i: (0, 0)),
          pl.BlockSpec((HH, H), lambda i: (0, 0)),
          pl.BlockSpec((1, H), lambda i: (0, 0)),
          pl.BlockSpec((1, H), lambda i: (0, 0)),
      ],
      out_specs=[
          pl.BlockSpec((2, BLK, HH), lambda i: (0, i, 0)),
          pl.BlockSpec((2, BLK), lambda i: (0, i)),
      ],
      out_shape=[
          jax.ShapeDtypeStruct((2, NP_, HH), _f32),
          jax.ShapeDtypeStruct((2, NP_), _f32),
      ],
  )(x[0], x[1], W[:HH], W[HH:], a_src.reshape(1, H), a_dst.reshape(1, H))


def _gat_epilogue(un, den, b, pos, rbf, last):
  def body(u_ref, d_ref, b_ref, p_ref, r_ref, o_ref):
    den = d_ref[0] + d_ref[1] + 1e-16
    for h in range(2):
      xh = u_ref[h] / den[:, None] + b_ref[0, h * HH:(h + 1) * HH]
      if not last:
        rb = r_ref[0, 0]
        mix = xh * (1.0 - ALPHA) + p_ref[h] * ALPHA
        xh = jnp.maximum(rb * mix + (1.0 - rb) * xh, 0.0)
      o_ref[h] = xh

  return pl.pallas_call(
      body,
      grid=(NP_ // BLK,),
      in_specs=[
          pl.BlockSpec((2, BLK, HH), lambda i: (0, i, 0)),
          pl.BlockSpec((2, BLK), lambda i: (0, i)),
          pl.BlockSpec((1, H), lambda i: (0, 0)),
          pl.BlockSpec((2, BLK, HH), lambda i: (0, i, 0)),
          pl.BlockSpec((1, 1), lambda i: (0, 0)),
      ],
      out_specs=pl.BlockSpec((2, BLK, HH), lambda i: (0, i, 0)),
      out_shape=jax.ShapeDtypeStruct((2, NP_, HH), _f32),
  )(un, den, b.reshape(1, H), pos, rbf)


def _pe_finish(h):
  def body(h_ref, o_ref):
    o_ref[...] = jnp.tanh(jnp.maximum(h_ref[...], 0.0))

  return pl.pallas_call(
      body,
      grid=(NP_ // BLK,),
      in_specs=[pl.BlockSpec((2, BLK, HH), lambda i: (0, i, 0))],
      out_specs=pl.BlockSpec((2, BLK, HH), lambda i: (0, i, 0)),
      out_shape=jax.ShapeDtypeStruct((2, NP_, HH), _f32),
  )(h)


def _final(x, W, b):
  def body(x0_ref, x1_ref, w0_ref, w1_ref, b_ref, e_ref, l_ref):
    e = (jnp.dot(x0_ref[...], w0_ref[...], preferred_element_type=_f32) +
         jnp.dot(x1_ref[...], w1_ref[...], preferred_element_type=_f32) +
         b_ref[...])
    m = jnp.max(e, axis=1, keepdims=True)
    z = e - m
    lse = jnp.log(jnp.sum(jnp.exp(z), axis=1, keepdims=True))
    e_ref[...] = e
    l_ref[...] = z - lse

  return pl.pallas_call(
      body,
      grid=(NP_ // BLK,),
      in_specs=[
          pl.BlockSpec((BLK, HH), lambda i: (i, 0)),
          pl.BlockSpec((BLK, HH), lambda i: (i, 0)),
          pl.BlockSpec((HH, C), lambda i: (0, 0)),
          pl.BlockSpec((HH, C), lambda i: (0, 0)),
          pl.BlockSpec((1, C), lambda i: (0, 0)),
      ],
      out_specs=[
          pl.BlockSpec((BLK, C), lambda i: (i, 0)),
          pl.BlockSpec((BLK, C), lambda i: (i, 0)),
      ],
      out_shape=[
          jax.ShapeDtypeStruct((NP_, C), _f32),
          jax.ShapeDtypeStruct((NP_, C), _f32),
      ],
  )(x[0], x[1], W[:HH], W[HH:], b.reshape(1, C))


EW = EG // (NSC * NSUB)


def _build_edge_logits():
  return functools.partial(
      pl.kernel,
      out_type=(
          jax.ShapeDtypeStruct((EG,), _f32),
          jax.ShapeDtypeStruct((NSC * NP_,), _f32),
      ),
      mesh=_mesh(),
      compiler_params=pltpu.CompilerParams(needs_layout_passes=False),
      scratch_types=[
          pltpu.VMEM((NP_,), _f32),
          pltpu.VMEM((NP_,), _f32),
          pltpu.VMEM((EW,), _i32),
          pltpu.VMEM((EW,), _i32),
          pltpu.VMEM((EW,), _f32),
          pltpu.VMEM((NP_,), _f32),
          pltpu.VMEM((RPN,), _f32),
          pltpu.VMEM((RPN,), _f32),
          pltpu.VMEM_SHARED((NSUB, NP_), _f32),
      ],
  )(_edge_logits_body)


def _edge_logits_body(asd_hbm, src_hbm, dst_hbm, ex_hbm, den_hbm,
                      as_v, ad_v, src_v, dst_v, ex_v, dp_v, acc_v, red_v,
                      shared):
  c = lax.axis_index("c")
  s = lax.axis_index("s")
  wid = s * NSC + c
  pltpu.sync_copy(asd_hbm.at[pl.ds(0, NP_)], as_v)
  pltpu.sync_copy(asd_hbm.at[pl.ds(NP_, NP_)], ad_v)
  pltpu.sync_copy(src_hbm.at[pl.ds(wid * EW, EW)], src_v)
  pltpu.sync_copy(dst_hbm.at[pl.ds(wid * EW, EW)], dst_v)

  def zero16(i, _):
    dp_v[pl.ds(i * 16, 16)] = jnp.zeros((16,), _f32)
    return 0
  lax.fori_loop(0, NP_ // 16, zero16, 0)

  def grp(g, _):
    for k in range(8):
      sl = pl.ds((g * 8 + k) * 16, 16)
      si = src_v[sl]
      di = dst_v[sl]
      ev = plsc.load_gather(as_v, [si]) + plsc.load_gather(ad_v, [di])
      ev = jnp.where(ev >= 0.0, ev, ev * NEG_SLOPE)
      ex = jnp.exp(ev)
      ex_v[sl] = ex
      plsc.addupdate_scatter(dp_v, [di], ex)
    return 0
  lax.fori_loop(0, EW // 128, grp, 0)

  pltpu.sync_copy(ex_v, ex_hbm.at[pl.ds(wid * EW, EW)])
  pltpu.sync_copy(dp_v, shared.at[s])
  plsc.subcore_barrier()

  def zacc(i, _):
    acc_v[pl.ds(i * 16, 16)] = jnp.zeros((16,), _f32)
    return 0
  lax.fori_loop(0, RPN // 16, zacc, 0)
  for i in range(NSUB):
    pltpu.sync_copy(shared.at[i, pl.ds(s * RPN, RPN)], red_v)

    def addv(j, _):
      sl = pl.ds(j * 16, 16)
      acc_v[sl] = acc_v[sl] + red_v[sl]
      return 0
    lax.fori_loop(0, RPN // 16, addv, 0)
  pltpu.sync_copy(acc_v, den_hbm.at[pl.ds(c * NP_ + s * RPN, RPN)])


CH = 8


def _make_wscatter(R):
  rpw = R // NSUB

  def wscatter(tbl_hbm, w_hbm, src_hbm, dst_hbm, out_hbm,
               src_v, dst_v, w_v, rows0, rows1, acc, sem0, sem1):
    c = lax.axis_index("c")
    s = lax.axis_index("s")

    def zrow(r, _):
      for k in range(8):
        rows0[r, pl.ds(k * 16, 16)] = jnp.zeros((16,), _f32)
      return 0
    lax.fori_loop(0, 128, zrow, 0)
    for i in range(RPN // 128):
      pltpu.sync_copy(rows0, acc.at[pl.ds(s * RPN + i * 128, 128)])

    plsc.subcore_barrier()

    def chunk(ci, _):
      base = s * rpw + ci * CH
      pltpu.sync_copy(src_hbm.at[pl.ds(base, CH)], src_v)
      pltpu.sync_copy(dst_hbm.at[pl.ds(base, CH)], dst_v)
      pltpu.sync_copy(w_hbm.at[pl.ds(base * 128, CH * 128)], w_v)

      def adj(r, _):
        for k in range(8):
          sl = pl.ds(k * 16, 16)
          src_v[r, sl] = src_v[r, sl] + c * NP_
        return 0
      lax.fori_loop(0, CH, adj, 0)

      def proc(buf, j):
        def scale(e, _):
          wb = plsc.load_gather(w_v, [jnp.full((16,), j * 128 + e, _i32)])
          for k in range(8):
            sl = pl.ds(k * 16, 16)
            buf[e, sl] = buf[e, sl] * wb
          return 0
        lax.fori_loop(0, 128, scale, 0)
        pltpu.sync_copy(buf, acc.at[dst_v.at[j]], add=True)

      pltpu.async_copy(tbl_hbm.at[src_v.at[0]], rows0, sem0)

      def pair(jj, _):
        r0 = 2 * jj
        pltpu.async_copy(tbl_hbm.at[src_v.at[r0 + 1]], rows1, sem1)
        pltpu.make_async_copy(tbl_hbm.at[src_v.at[r0]], rows0, sem0).wait()
        proc(rows0, r0)

        @pl.when(jj < CH // 2 - 1)
        def _():
          pltpu.async_copy(tbl_hbm.at[src_v.at[r0 + 2]], rows0, sem0)
        pltpu.make_async_copy(tbl_hbm.at[src_v.at[r0 + 1]], rows1, sem1).wait()
        proc(rows1, r0 + 1)
        return 0
      lax.fori_loop(0, CH // 2, pair, 0)
      return 0
    lax.fori_loop(0, rpw // CH, chunk, 0)

    plsc.subcore_barrier()
    pltpu.sync_copy(acc.at[pl.ds(s * RPN, RPN)],
                    out_hbm.at[pl.ds(c * NP_ + s * RPN, RPN)])

  return functools.partial(
      pl.kernel,
      out_type=jax.ShapeDtypeStruct((NSC * NP_, HH), _f32),
      mesh=_mesh(),
      compiler_params=pltpu.CompilerParams(needs_layout_passes=False),
      scratch_types=[
          pltpu.VMEM((CH, 128), _i32),
          pltpu.VMEM((CH, 128), _i32),
          pltpu.VMEM((CH * 128,), _f32),
          pltpu.VMEM((128, HH), _f32),
          pltpu.VMEM((128, HH), _f32),
          pltpu.VMEM_SHARED((NP_, HH), _f32),
          pltpu.SemaphoreType.DMA,
          pltpu.SemaphoreType.DMA,
      ],
  )(wscatter)


def kernel(x_h, adj, edge_index, pos_feat, run_base, W_pos, b_pos, W_pe, b_pe,
           W_init, b_init, W_gat, att_src, att_dst, b_gat, W_last, b_last):
  ei = edge_index.astype(_i32)
  loops = jnp.arange(N, dtype=_i32)
  padg = N + jnp.arange(EG - E - N, dtype=_i32) % (NP_ - N)
  src_g = jnp.concatenate([ei[0], loops, padg]).reshape(RG, 128)
  dst_g = jnp.concatenate([ei[1], loops, padg]).reshape(RG, 128)
  pads = N + jnp.arange(EG - E, dtype=_i32) % (NP_ - N)
  src_s = jnp.concatenate([ei[0], pads]).reshape(RG, 128)
  dst_s = jnp.concatenate([ei[1], pads]).reshape(RG, 128)
  w_s = jnp.concatenate([adj, jnp.zeros((EG - E,), _f32)])

  x_h_p = jnp.pad(x_h, ((0, NP_ - N), (0, 0)))
  pos_p = jnp.pad(pos_feat, ((0, NP_ - N), (0, 0)))
  rbf = jnp.asarray(jnp.asarray(run_base) == 0, _f32).reshape(1, 1)

  _edge_logits = _build_edge_logits()
  _wscatter = _make_wscatter(RG)

  h = _mm_split([pos_p], [W_pos], b_pos)
  for i in range(L_PE):
    h = _mm_split([h[0], h[1]], [W_pe[i][:HH], W_pe[i][HH:]], b_pe[i],
                  relu_in=(i > 0))
    h = _wscatter(h.reshape(NSC * NP_, HH), w_s, src_s, dst_s)
    h = h.reshape(2, NP_, HH)
  pos_split = _pe_finish(h)

  x = _mm_split([x_h_p], [W_init], b_init)
  for i in range(L_GCN):
    xw, a_sd = _gat_pre(x, W_gat[i], att_src[i], att_dst[i])
    ex, den = _edge_logits(a_sd.reshape(NSC * NP_), src_g.reshape(EG),
                           dst_g.reshape(EG))
    un = _wscatter(xw.reshape(NSC * NP_, HH), ex, src_g, dst_g)
    x = _gat_epilogue(un.reshape(2, NP_, HH), den.reshape(2, NP_), b_gat[i],
                      pos_split, rbf, last=(i == L_GCN - 1))

  emb, logp = _final(x, W_last, b_last)
  return emb[:N], logp[:N]


# speedup vs baseline: 13.2873x; 1.0138x over previous
import functools

import jax
import jax.numpy as jnp
from jax import lax
from jax.experimental import pallas as pl
from jax.experimental.pallas import tpu as pltpu
from jax.experimental.pallas import tpu_sc as plsc

N = 10000
E = 160000
D_IN = 256
H = 256
HH = 128
PE_DIM = 16
C = 40
L_GCN = 4
L_PE = 2
ALPHA = 0.1
NEG_SLOPE = 0.2

NP_ = 10240
EG = 180224
RG = EG // 128
BLK = 512
NSC = 2
NSUB = 16
RPN = NP_ // NSUB

def _mesh():
  return plsc.VectorSubcoreMesh(
      core_axis_name="c", subcore_axis_name="s",
      num_cores=NSC, num_subcores=NSUB)

_f32 = jnp.float32
_i32 = jnp.int32


def _mm_split(xs, Ws, b, relu_in=False):
  nx = len(xs)

  def body(*refs):
    x_refs = refs[:nx]
    w_refs = refs[nx:2 * nx]
    b_ref = refs[2 * nx]
    o_ref = refs[2 * nx + 1]
    acc = None
    for xr, wr in zip(x_refs, w_refs):
      xv = xr[...]
      if relu_in:
        xv = jnp.maximum(xv, 0.0)
      p = jnp.dot(xv, wr[...], preferred_element_type=_f32)
      acc = p if acc is None else acc + p
    acc = acc + b_ref[...]
    o_ref[0] = acc[:, :HH]
    o_ref[1] = acc[:, HH:]

  in_specs = []
  for x in xs:
    k = x.shape[1]
    in_specs.append(pl.BlockSpec((BLK, k), lambda i: (i, 0)))
  for w in Ws:
    in_specs.append(pl.BlockSpec(w.shape, lambda i: (0, 0)))
  in_specs.append(pl.BlockSpec((1, H), lambda i: (0, 0)))
  return pl.pallas_call(
      body,
      grid=(NP_ // BLK,),
      in_specs=in_specs,
      out_specs=pl.BlockSpec((2, BLK, HH), lambda i: (0, i, 0)),
      out_shape=jax.ShapeDtypeStruct((2, NP_, HH), _f32),
  )(*xs, *Ws, b.reshape(1, H))


def _gat_pre(x, W, a_src, a_dst):
  def body(x0_ref, x1_ref, w0_ref, w1_ref, as_ref, ad_ref, xw_ref, asd_ref):
    xw = (jnp.dot(x0_ref[...], w0_ref[...], preferred_element_type=_f32) +
          jnp.dot(x1_ref[...], w1_ref[...], preferred_element_type=_f32))
    xw_ref[0] = xw[:, :HH]
    xw_ref[1] = xw[:, HH:]
    asd_ref[0] = jnp.sum(xw * as_ref[...], axis=1)
    asd_ref[1] = jnp.sum(xw * ad_ref[...], axis=1)

  return pl.pallas_call(
      body,
      grid=(NP_ // BLK,),
      in_specs=[
          pl.BlockSpec((BLK, HH), lambda i: (i, 0)),
          pl.BlockSpec((BLK, HH), lambda i: (i, 0)),
          pl.BlockSpec((HH, H), lambda i: (0, 0)),
          pl.BlockSpec((HH, H), lambda i: (0, 0)),
          pl.BlockSpec((1, H), lambda i: (0, 0)),
          pl.BlockSpec((1, H), lambda i: (0, 0)),
      ],
      out_specs=[
          pl.BlockSpec((2, BLK, HH), lambda i: (0, i, 0)),
          pl.BlockSpec((2, BLK), lambda i: (0, i)),
      ],
      out_shape=[
          jax.ShapeDtypeStruct((2, NP_, HH), _f32),
          jax.ShapeDtypeStruct((2, NP_), _f32),
      ],
  )(x[0], x[1], W[:HH], W[HH:], a_src.reshape(1, H), a_dst.reshape(1, H))


def _gat_epilogue(un, den, b, pos, rbf, last):
  def body(u_ref, d_ref, b_ref, p_ref, r_ref, o_ref):
    den = d_ref[0] + d_ref[1] + 1e-16
    for h in range(2):
      xh = u_ref[h] / den[:, None] + b_ref[0, h * HH:(h + 1) * HH]
      if not last:
        rb = r_ref[0, 0]
        mix = xh * (1.0 - ALPHA) + p_ref[h] * ALPHA
        xh = jnp.maximum(rb * mix + (1.0 - rb) * xh, 0.0)
      o_ref[h] = xh

  return pl.pallas_call(
      body,
      grid=(NP_ // BLK,),
      in_specs=[
          pl.BlockSpec((2, BLK, HH), lambda i: (0, i, 0)),
          pl.BlockSpec((2, BLK), lambda i: (0, i)),
          pl.BlockSpec((1, H), lambda i: (0, 0)),
          pl.BlockSpec((2, BLK, HH), lambda i: (0, i, 0)),
          pl.BlockSpec((1, 1), lambda i: (0, 0)),
      ],
      out_specs=pl.BlockSpec((2, BLK, HH), lambda i: (0, i, 0)),
      out_shape=jax.ShapeDtypeStruct((2, NP_, HH), _f32),
  )(un, den, b.reshape(1, H), pos, rbf)


def _pe_finish(h):
  def body(h_ref, o_ref):
    o_ref[...] = jnp.tanh(jnp.maximum(h_ref[...], 0.0))

  return pl.pallas_call(
      body,
      grid=(NP_ // BLK,),
      in_specs=[pl.BlockSpec((2, BLK, HH), lambda i: (0, i, 0))],
      out_specs=pl.BlockSpec((2, BLK, HH), lambda i: (0, i, 0)),
      out_shape=jax.ShapeDtypeStruct((2, NP_, HH), _f32),
  )(h)


def _final(x, W, b):
  def body(x0_ref, x1_ref, w0_ref, w1_ref, b_ref, e_ref, l_ref):
    e = (jnp.dot(x0_ref[...], w0_ref[...], preferred_element_type=_f32) +
         jnp.dot(x1_ref[...], w1_ref[...], preferred_element_type=_f32) +
         b_ref[...])
    m = jnp.max(e, axis=1, keepdims=True)
    z = e - m
    lse = jnp.log(jnp.sum(jnp.exp(z), axis=1, keepdims=True))
    e_ref[...] = e
    l_ref[...] = z - lse

  return pl.pallas_call(
      body,
      grid=(NP_ // BLK,),
      in_specs=[
          pl.BlockSpec((BLK, HH), lambda i: (i, 0)),
          pl.BlockSpec((BLK, HH), lambda i: (i, 0)),
          pl.BlockSpec((HH, C), lambda i: (0, 0)),
          pl.BlockSpec((HH, C), lambda i: (0, 0)),
          pl.BlockSpec((1, C), lambda i: (0, 0)),
      ],
      out_specs=[
          pl.BlockSpec((BLK, C), lambda i: (i, 0)),
          pl.BlockSpec((BLK, C), lambda i: (i, 0)),
      ],
      out_shape=[
          jax.ShapeDtypeStruct((NP_, C), _f32),
          jax.ShapeDtypeStruct((NP_, C), _f32),
      ],
  )(x[0], x[1], W[:HH], W[HH:], b.reshape(1, C))


EW = EG // (NSC * NSUB)


def _build_edge_logits():
  return functools.partial(
      pl.kernel,
      out_type=(
          jax.ShapeDtypeStruct((EG,), _f32),
          jax.ShapeDtypeStruct((NSC * NP_,), _f32),
      ),
      mesh=_mesh(),
      compiler_params=pltpu.CompilerParams(needs_layout_passes=False),
      scratch_types=[
          pltpu.VMEM((NP_,), _f32),
          pltpu.VMEM((NP_,), _f32),
          pltpu.VMEM((EW,), _i32),
          pltpu.VMEM((EW,), _i32),
          pltpu.VMEM((EW,), _f32),
          pltpu.VMEM((NP_,), _f32),
          pltpu.VMEM((RPN,), _f32),
          pltpu.VMEM((RPN,), _f32),
          pltpu.VMEM_SHARED((NSUB, NP_), _f32),
      ],
  )(_edge_logits_body)


def _edge_logits_body(asd_hbm, src_hbm, dst_hbm, ex_hbm, den_hbm,
                      as_v, ad_v, src_v, dst_v, ex_v, dp_v, acc_v, red_v,
                      shared):
  c = lax.axis_index("c")
  s = lax.axis_index("s")
  wid = s * NSC + c
  pltpu.sync_copy(asd_hbm.at[pl.ds(0, NP_)], as_v)
  pltpu.sync_copy(asd_hbm.at[pl.ds(NP_, NP_)], ad_v)
  pltpu.sync_copy(src_hbm.at[pl.ds(wid * EW, EW)], src_v)
  pltpu.sync_copy(dst_hbm.at[pl.ds(wid * EW, EW)], dst_v)

  def zero16(i, _):
    dp_v[pl.ds(i * 16, 16)] = jnp.zeros((16,), _f32)
    return 0
  lax.fori_loop(0, NP_ // 16, zero16, 0)

  def grp(g, _):
    for k in range(8):
      sl = pl.ds((g * 8 + k) * 16, 16)
      si = src_v[sl]
      di = dst_v[sl]
      ev = plsc.load_gather(as_v, [si]) + plsc.load_gather(ad_v, [di])
      ev = jnp.where(ev >= 0.0, ev, ev * NEG_SLOPE)
      ex = jnp.exp(ev)
      ex_v[sl] = ex
      plsc.addupdate_scatter(dp_v, [di], ex)
    return 0
  lax.fori_loop(0, EW // 128, grp, 0)

  pltpu.sync_copy(ex_v, ex_hbm.at[pl.ds(wid * EW, EW)])
  pltpu.sync_copy(dp_v, shared.at[s])
  plsc.subcore_barrier()

  def zacc(i, _):
    acc_v[pl.ds(i * 16, 16)] = jnp.zeros((16,), _f32)
    return 0
  lax.fori_loop(0, RPN // 16, zacc, 0)
  for i in range(NSUB):
    pltpu.sync_copy(shared.at[i, pl.ds(s * RPN, RPN)], red_v)

    def addv(j, _):
      sl = pl.ds(j * 16, 16)
      acc_v[sl] = acc_v[sl] + red_v[sl]
      return 0
    lax.fori_loop(0, RPN // 16, addv, 0)
  pltpu.sync_copy(acc_v, den_hbm.at[pl.ds(c * NP_ + s * RPN, RPN)])


CH = 8


def _make_wscatter(R):
  rpw = R // NSUB

  def wscatter(tbl_hbm, w_hbm, src_hbm, dst_hbm, out_hbm,
               src_v, dst_v, w_v, rows0, rows1, acc, sem0, sem1, semS0,
               semS1):
    c = lax.axis_index("c")
    s = lax.axis_index("s")

    def zrow(r, _):
      for k in range(8):
        rows0[r, pl.ds(k * 16, 16)] = jnp.zeros((16,), _f32)
      return 0
    lax.fori_loop(0, 128, zrow, 0)
    for i in range(RPN // 128):
      pltpu.sync_copy(rows0, acc.at[pl.ds(s * RPN + i * 128, 128)])

    plsc.subcore_barrier()

    def chunk(ci, _):
      @pl.when(ci > 0)
      def _():
        pltpu.make_async_copy(rows1, acc.at[dst_v.at[CH - 1]], semS1).wait()
      base = s * rpw + ci * CH
      pltpu.sync_copy(src_hbm.at[pl.ds(base, CH)], src_v)
      pltpu.sync_copy(dst_hbm.at[pl.ds(base, CH)], dst_v)
      pltpu.sync_copy(w_hbm.at[pl.ds(base * 128, CH * 128)], w_v)

      def adj(r, _):
        for k in range(8):
          sl = pl.ds(k * 16, 16)
          src_v[r, sl] = src_v[r, sl] + c * NP_
        return 0
      lax.fori_loop(0, CH, adj, 0)

      def scale(buf, j):
        def step(e, _):
          wb = plsc.load_gather(w_v, [jnp.full((16,), j * 128 + e, _i32)])
          for k in range(8):
            sl = pl.ds(k * 16, 16)
            buf[e, sl] = buf[e, sl] * wb
          return 0
        lax.fori_loop(0, 128, step, 0)

      pltpu.async_copy(tbl_hbm.at[src_v.at[0]], rows0, sem0)

      def pair(jj, _):
        r0 = 2 * jj

        @pl.when(jj > 0)
        def _():
          pltpu.make_async_copy(rows1, acc.at[dst_v.at[r0]], semS1).wait()
        pltpu.async_copy(tbl_hbm.at[src_v.at[r0 + 1]], rows1, sem1)
        pltpu.make_async_copy(tbl_hbm.at[src_v.at[r0]], rows0, sem0).wait()
        scale(rows0, r0)
        sc0 = pltpu.async_copy(rows0, acc.at[dst_v.at[r0]], semS0, add=True)
        pltpu.make_async_copy(tbl_hbm.at[src_v.at[r0 + 1]], rows1, sem1).wait()
        scale(rows1, r0 + 1)
        sc0.wait()

        @pl.when(jj < CH // 2 - 1)
        def _():
          pltpu.async_copy(tbl_hbm.at[src_v.at[r0 + 2]], rows0, sem0)
        pltpu.async_copy(rows1, acc.at[dst_v.at[r0 + 1]], semS1, add=True)
        return 0
      lax.fori_loop(0, CH // 2, pair, 0)
      return 0
    lax.fori_loop(0, rpw // CH, chunk, 0)
    pltpu.make_async_copy(rows1, acc.at[dst_v.at[CH - 1]], semS1).wait()

    plsc.subcore_barrier()
    pltpu.sync_copy(acc.at[pl.ds(s * RPN, RPN)],
                    out_hbm.at[pl.ds(c * NP_ + s * RPN, RPN)])

  return functools.partial(
      pl.kernel,
      out_type=jax.ShapeDtypeStruct((NSC * NP_, HH), _f32),
      mesh=_mesh(),
      compiler_params=pltpu.CompilerParams(needs_layout_passes=False),
      scratch_types=[
          pltpu.VMEM((CH, 128), _i32),
          pltpu.VMEM((CH, 128), _i32),
          pltpu.VMEM((CH * 128,), _f32),
          pltpu.VMEM((128, HH), _f32),
          pltpu.VMEM((128, HH), _f32),
          pltpu.VMEM_SHARED((NP_, HH), _f32),
          pltpu.SemaphoreType.DMA,
          pltpu.SemaphoreType.DMA,
          pltpu.SemaphoreType.DMA,
          pltpu.SemaphoreType.DMA,
      ],
  )(wscatter)


def kernel(x_h, adj, edge_index, pos_feat, run_base, W_pos, b_pos, W_pe, b_pe,
           W_init, b_init, W_gat, att_src, att_dst, b_gat, W_last, b_last):
  ei = edge_index.astype(_i32)
  loops = jnp.arange(N, dtype=_i32)
  padg = N + jnp.arange(EG - E - N, dtype=_i32) % (NP_ - N)
  src_g = jnp.concatenate([ei[0], loops, padg]).reshape(RG, 128)
  dst_g = jnp.concatenate([ei[1], loops, padg]).reshape(RG, 128)
  pads = N + jnp.arange(EG - E, dtype=_i32) % (NP_ - N)
  src_s = jnp.concatenate([ei[0], pads]).reshape(RG, 128)
  dst_s = jnp.concatenate([ei[1], pads]).reshape(RG, 128)
  w_s = jnp.concatenate([adj, jnp.zeros((EG - E,), _f32)])

  x_h_p = jnp.pad(x_h, ((0, NP_ - N), (0, 0)))
  pos_p = jnp.pad(pos_feat, ((0, NP_ - N), (0, 0)))
  rbf = jnp.asarray(jnp.asarray(run_base) == 0, _f32).reshape(1, 1)

  _edge_logits = _build_edge_logits()
  _wscatter = _make_wscatter(RG)

  h = _mm_split([pos_p], [W_pos], b_pos)
  for i in range(L_PE):
    h = _mm_split([h[0], h[1]], [W_pe[i][:HH], W_pe[i][HH:]], b_pe[i],
                  relu_in=(i > 0))
    h = _wscatter(h.reshape(NSC * NP_, HH), w_s, src_s, dst_s)
    h = h.reshape(2, NP_, HH)
  pos_split = _pe_finish(h)

  x = _mm_split([x_h_p], [W_init], b_init)
  for i in range(L_GCN):
    xw, a_sd = _gat_pre(x, W_gat[i], att_src[i], att_dst[i])
    ex, den = _edge_logits(a_sd.reshape(NSC * NP_), src_g.reshape(EG),
                           dst_g.reshape(EG))
    un = _wscatter(xw.reshape(NSC * NP_, HH), ex, src_g, dst_g)
    x = _gat_epilogue(un.reshape(2, NP_, HH), den.reshape(2, NP_), b_gat[i],
                      pos_split, rbf, last=(i == L_GCN - 1))

  emb, logp = _final(x, W_last, b_last)
  return emb[:N], logp[:N]


# speedup vs baseline: 15.5209x; 1.1681x over previous
import functools

import jax
import jax.numpy as jnp
from jax import lax
from jax.experimental import pallas as pl
from jax.experimental.pallas import tpu as pltpu
from jax.experimental.pallas import tpu_sc as plsc

N = 10000
E = 160000
D_IN = 256
H = 256
HH = 128
PE_DIM = 16
C = 40
L_GCN = 4
L_PE = 2
ALPHA = 0.1
NEG_SLOPE = 0.2

NP_ = 10240
EG = 180224
RG = EG // 128
BLK = 512
NSC = 2
NSUB = 16
RPN = NP_ // NSUB

def _mesh():
  return plsc.VectorSubcoreMesh(
      core_axis_name="c", subcore_axis_name="s",
      num_cores=NSC, num_subcores=NSUB)

_f32 = jnp.float32
_i32 = jnp.int32


def _mm_split(xs, Ws, b, relu_in=False):
  nx = len(xs)

  def body(*refs):
    x_refs = refs[:nx]
    w_refs = refs[nx:2 * nx]
    b_ref = refs[2 * nx]
    o_ref = refs[2 * nx + 1]
    acc = None
    for xr, wr in zip(x_refs, w_refs):
      xv = xr[...]
      if relu_in:
        xv = jnp.maximum(xv, 0.0)
      p = jnp.dot(xv, wr[...], preferred_element_type=_f32)
      acc = p if acc is None else acc + p
    acc = acc + b_ref[...]
    o_ref[0] = acc[:, :HH]
    o_ref[1] = acc[:, HH:]

  in_specs = []
  for x in xs:
    k = x.shape[1]
    in_specs.append(pl.BlockSpec((BLK, k), lambda i: (i, 0)))
  for w in Ws:
    in_specs.append(pl.BlockSpec(w.shape, lambda i: (0, 0)))
  in_specs.append(pl.BlockSpec((1, H), lambda i: (0, 0)))
  return pl.pallas_call(
      body,
      grid=(NP_ // BLK,),
      in_specs=in_specs,
      out_specs=pl.BlockSpec((2, BLK, HH), lambda i: (0, i, 0)),
      out_shape=jax.ShapeDtypeStruct((2, NP_, HH), _f32),
  )(*xs, *Ws, b.reshape(1, H))


def _gat_pre(x, W, a_src, a_dst):
  def body(x0_ref, x1_ref, w0_ref, w1_ref, as_ref, ad_ref, xw_ref, asd_ref):
    xw = (jnp.dot(x0_ref[...], w0_ref[...], preferred_element_type=_f32) +
          jnp.dot(x1_ref[...], w1_ref[...], preferred_element_type=_f32))
    xw_ref[0] = xw[:, :HH]
    xw_ref[1] = xw[:, HH:]
    asd_ref[0] = jnp.sum(xw * as_ref[...], axis=1)
    asd_ref[1] = jnp.sum(xw * ad_ref[...], axis=1)

  return pl.pallas_call(
      body,
      grid=(NP_ // BLK,),
      in_specs=[
          pl.BlockSpec((BLK, HH), lambda i: (i, 0)),
          pl.BlockSpec((BLK, HH), lambda i: (i, 0)),
          pl.BlockSpec((HH, H), lambda i: (0, 0)),
          pl.BlockSpec((HH, H), lambda i: (0, 0)),
          pl.BlockSpec((1, H), lambda i: (0, 0)),
          pl.BlockSpec((1, H), lambda i: (0, 0)),
      ],
      out_specs=[
          pl.BlockSpec((2, BLK, HH), lambda i: (0, i, 0)),
          pl.BlockSpec((2, BLK), lambda i: (0, i)),
      ],
      out_shape=[
          jax.ShapeDtypeStruct((2, NP_, HH), _f32),
          jax.ShapeDtypeStruct((2, NP_), _f32),
      ],
  )(x[0], x[1], W[:HH], W[HH:], a_src.reshape(1, H), a_dst.reshape(1, H))


def _gat_epilogue(un, den, b, pos, rbf, last):
  def body(u_ref, d_ref, b_ref, p_ref, r_ref, o_ref):
    den = d_ref[0] + d_ref[1] + 1e-16
    for h in range(2):
      xh = u_ref[h] / den[:, None] + b_ref[0, h * HH:(h + 1) * HH]
      if not last:
        rb = r_ref[0, 0]
        mix = xh * (1.0 - ALPHA) + p_ref[h] * ALPHA
        xh = jnp.maximum(rb * mix + (1.0 - rb) * xh, 0.0)
      o_ref[h] = xh

  return pl.pallas_call(
      body,
      grid=(NP_ // BLK,),
      in_specs=[
          pl.BlockSpec((2, BLK, HH), lambda i: (0, i, 0)),
          pl.BlockSpec((2, BLK), lambda i: (0, i)),
          pl.BlockSpec((1, H), lambda i: (0, 0)),
          pl.BlockSpec((2, BLK, HH), lambda i: (0, i, 0)),
          pl.BlockSpec((1, 1), lambda i: (0, 0)),
      ],
      out_specs=pl.BlockSpec((2, BLK, HH), lambda i: (0, i, 0)),
      out_shape=jax.ShapeDtypeStruct((2, NP_, HH), _f32),
  )(un, den, b.reshape(1, H), pos, rbf)


def _pe_finish(h):
  def body(h_ref, o_ref):
    o_ref[...] = jnp.tanh(jnp.maximum(h_ref[...], 0.0))

  return pl.pallas_call(
      body,
      grid=(NP_ // BLK,),
      in_specs=[pl.BlockSpec((2, BLK, HH), lambda i: (0, i, 0))],
      out_specs=pl.BlockSpec((2, BLK, HH), lambda i: (0, i, 0)),
      out_shape=jax.ShapeDtypeStruct((2, NP_, HH), _f32),
  )(h)


def _final(x, W, b):
  def body(x0_ref, x1_ref, w0_ref, w1_ref, b_ref, e_ref, l_ref):
    e = (jnp.dot(x0_ref[...], w0_ref[...], preferred_element_type=_f32) +
         jnp.dot(x1_ref[...], w1_ref[...], preferred_element_type=_f32) +
         b_ref[...])
    m = jnp.max(e, axis=1, keepdims=True)
    z = e - m
    lse = jnp.log(jnp.sum(jnp.exp(z), axis=1, keepdims=True))
    e_ref[...] = e
    l_ref[...] = z - lse

  return pl.pallas_call(
      body,
      grid=(NP_ // BLK,),
      in_specs=[
          pl.BlockSpec((BLK, HH), lambda i: (i, 0)),
          pl.BlockSpec((BLK, HH), lambda i: (i, 0)),
          pl.BlockSpec((HH, C), lambda i: (0, 0)),
          pl.BlockSpec((HH, C), lambda i: (0, 0)),
          pl.BlockSpec((1, C), lambda i: (0, 0)),
      ],
      out_specs=[
          pl.BlockSpec((BLK, C), lambda i: (i, 0)),
          pl.BlockSpec((BLK, C), lambda i: (i, 0)),
      ],
      out_shape=[
          jax.ShapeDtypeStruct((NP_, C), _f32),
          jax.ShapeDtypeStruct((NP_, C), _f32),
      ],
  )(x[0], x[1], W[:HH], W[HH:], b.reshape(1, C))


EW = EG // (NSC * NSUB)


def _build_edge_logits():
  return functools.partial(
      pl.kernel,
      out_type=(
          jax.ShapeDtypeStruct((EG,), _f32),
          jax.ShapeDtypeStruct((NSC * NP_,), _f32),
      ),
      mesh=_mesh(),
      compiler_params=pltpu.CompilerParams(needs_layout_passes=False),
      scratch_types=[
          pltpu.VMEM((NP_,), _f32),
          pltpu.VMEM((NP_,), _f32),
          pltpu.VMEM((EW,), _i32),
          pltpu.VMEM((EW,), _i32),
          pltpu.VMEM((EW,), _f32),
          pltpu.VMEM((NP_,), _f32),
          pltpu.VMEM((RPN,), _f32),
          pltpu.VMEM((RPN,), _f32),
          pltpu.VMEM_SHARED((NSUB, NP_), _f32),
      ],
  )(_edge_logits_body)


def _edge_logits_body(asd_hbm, src_hbm, dst_hbm, ex_hbm, den_hbm,
                      as_v, ad_v, src_v, dst_v, ex_v, dp_v, acc_v, red_v,
                      shared):
  c = lax.axis_index("c")
  s = lax.axis_index("s")
  wid = s * NSC + c
  pltpu.sync_copy(asd_hbm.at[pl.ds(0, NP_)], as_v)
  pltpu.sync_copy(asd_hbm.at[pl.ds(NP_, NP_)], ad_v)
  pltpu.sync_copy(src_hbm.at[pl.ds(wid * EW, EW)], src_v)
  pltpu.sync_copy(dst_hbm.at[pl.ds(wid * EW, EW)], dst_v)

  def zero16(i, _):
    dp_v[pl.ds(i * 16, 16)] = jnp.zeros((16,), _f32)
    return 0
  lax.fori_loop(0, NP_ // 16, zero16, 0)

  def grp(g, _):
    for k in range(8):
      sl = pl.ds((g * 8 + k) * 16, 16)
      si = src_v[sl]
      di = dst_v[sl]
      ev = plsc.load_gather(as_v, [si]) + plsc.load_gather(ad_v, [di])
      ev = jnp.where(ev >= 0.0, ev, ev * NEG_SLOPE)
      ex = jnp.exp(ev)
      ex_v[sl] = ex
      plsc.addupdate_scatter(dp_v, [di], ex)
    return 0
  lax.fori_loop(0, EW // 128, grp, 0)

  pltpu.sync_copy(ex_v, ex_hbm.at[pl.ds(wid * EW, EW)])
  pltpu.sync_copy(dp_v, shared.at[s])
  plsc.subcore_barrier()

  def zacc(i, _):
    acc_v[pl.ds(i * 16, 16)] = jnp.zeros((16,), _f32)
    return 0
  lax.fori_loop(0, RPN // 16, zacc, 0)
  for i in range(NSUB):
    pltpu.sync_copy(shared.at[i, pl.ds(s * RPN, RPN)], red_v)

    def addv(j, _):
      sl = pl.ds(j * 16, 16)
      acc_v[sl] = acc_v[sl] + red_v[sl]
      return 0
    lax.fori_loop(0, RPN // 16, addv, 0)
  pltpu.sync_copy(acc_v, den_hbm.at[pl.ds(c * NP_ + s * RPN, RPN)])


CH = 8


def _make_wscatter(R):
  rpw = R // NSUB

  def wscatter(tbl_hbm, w_hbm, src_hbm, dst_hbm, out_hbm,
               src_v, dst_v, w_v, rows0, rows1, acc, sem0, sem1, semS0,
               semS1):
    c = lax.axis_index("c")
    s = lax.axis_index("s")

    def zrow(r, _):
      for k in range(8):
        rows0[r, pl.ds(k * 16, 16)] = jnp.zeros((16,), _f32)
      return 0
    lax.fori_loop(0, 128, zrow, 0)
    for i in range(RPN // 128):
      pltpu.sync_copy(rows0, acc.at[pl.ds(s * RPN + i * 128, 128)])

    plsc.subcore_barrier()

    def chunk(ci, _):
      @pl.when(ci > 0)
      def _():
        pltpu.make_async_copy(rows1, acc.at[dst_v.at[CH - 1]], semS1).wait()
      base = s * rpw + ci * CH
      pltpu.sync_copy(src_hbm.at[pl.ds(base, CH)], src_v)
      pltpu.sync_copy(dst_hbm.at[pl.ds(base, CH)], dst_v)
      pltpu.sync_copy(w_hbm.at[pl.ds(base * 128, CH * 128)], w_v)

      def adj(r, _):
        for k in range(8):
          sl = pl.ds(k * 16, 16)
          src_v[r, sl] = src_v[r, sl] + c * NP_
        return 0
      lax.fori_loop(0, CH, adj, 0)

      def scale(buf, j):
        @plsc.parallel_loop(0, 128, step=1, unroll=4)
        def _(e):
          wb = plsc.load_gather(w_v, [jnp.full((16,), j * 128 + e, _i32)])
          for k in range(8):
            sl = pl.ds(k * 16, 16)
            buf[e, sl] = buf[e, sl] * wb

      pltpu.async_copy(tbl_hbm.at[src_v.at[0]], rows0, sem0)

      def pair(jj, _):
        r0 = 2 * jj

        @pl.when(jj > 0)
        def _():
          pltpu.make_async_copy(rows1, acc.at[dst_v.at[r0]], semS1).wait()
        pltpu.async_copy(tbl_hbm.at[src_v.at[r0 + 1]], rows1, sem1)
        pltpu.make_async_copy(tbl_hbm.at[src_v.at[r0]], rows0, sem0).wait()
        scale(rows0, r0)
        sc0 = pltpu.async_copy(rows0, acc.at[dst_v.at[r0]], semS0, add=True)
        pltpu.make_async_copy(tbl_hbm.at[src_v.at[r0 + 1]], rows1, sem1).wait()
        scale(rows1, r0 + 1)
        sc0.wait()

        @pl.when(jj < CH // 2 - 1)
        def _():
          pltpu.async_copy(tbl_hbm.at[src_v.at[r0 + 2]], rows0, sem0)
        pltpu.async_copy(rows1, acc.at[dst_v.at[r0 + 1]], semS1, add=True)
        return 0
      lax.fori_loop(0, CH // 2, pair, 0)
      return 0
    lax.fori_loop(0, rpw // CH, chunk, 0)
    pltpu.make_async_copy(rows1, acc.at[dst_v.at[CH - 1]], semS1).wait()

    plsc.subcore_barrier()
    pltpu.sync_copy(acc.at[pl.ds(s * RPN, RPN)],
                    out_hbm.at[pl.ds(c * NP_ + s * RPN, RPN)])

  return functools.partial(
      pl.kernel,
      out_type=jax.ShapeDtypeStruct((NSC * NP_, HH), _f32),
      mesh=_mesh(),
      compiler_params=pltpu.CompilerParams(needs_layout_passes=False),
      scratch_types=[
          pltpu.VMEM((CH, 128), _i32),
          pltpu.VMEM((CH, 128), _i32),
          pltpu.VMEM((CH * 128,), _f32),
          pltpu.VMEM((128, HH), _f32),
          pltpu.VMEM((128, HH), _f32),
          pltpu.VMEM_SHARED((NP_, HH), _f32),
          pltpu.SemaphoreType.DMA,
          pltpu.SemaphoreType.DMA,
          pltpu.SemaphoreType.DMA,
          pltpu.SemaphoreType.DMA,
      ],
  )(wscatter)


def kernel(x_h, adj, edge_index, pos_feat, run_base, W_pos, b_pos, W_pe, b_pe,
           W_init, b_init, W_gat, att_src, att_dst, b_gat, W_last, b_last):
  ei = edge_index.astype(_i32)
  loops = jnp.arange(N, dtype=_i32)
  padg = N + jnp.arange(EG - E - N, dtype=_i32) % (NP_ - N)
  src_g = jnp.concatenate([ei[0], loops, padg]).reshape(RG, 128)
  dst_g = jnp.concatenate([ei[1], loops, padg]).reshape(RG, 128)
  pads = N + jnp.arange(EG - E, dtype=_i32) % (NP_ - N)
  src_s = jnp.concatenate([ei[0], pads]).reshape(RG, 128)
  dst_s = jnp.concatenate([ei[1], pads]).reshape(RG, 128)
  w_s = jnp.concatenate([adj, jnp.zeros((EG - E,), _f32)])

  x_h_p = jnp.pad(x_h, ((0, NP_ - N), (0, 0)))
  pos_p = jnp.pad(pos_feat, ((0, NP_ - N), (0, 0)))
  rbf = jnp.asarray(jnp.asarray(run_base) == 0, _f32).reshape(1, 1)

  _edge_logits = _build_edge_logits()
  _wscatter = _make_wscatter(RG)

  h = _mm_split([pos_p], [W_pos], b_pos)
  for i in range(L_PE):
    h = _mm_split([h[0], h[1]], [W_pe[i][:HH], W_pe[i][HH:]], b_pe[i],
                  relu_in=(i > 0))
    h = _wscatter(h.reshape(NSC * NP_, HH), w_s, src_s, dst_s)
    h = h.reshape(2, NP_, HH)
  pos_split = _pe_finish(h)

  x = _mm_split([x_h_p], [W_init], b_init)
  for i in range(L_GCN):
    xw, a_sd = _gat_pre(x, W_gat[i], att_src[i], att_dst[i])
    ex, den = _edge_logits(a_sd.reshape(NSC * NP_), src_g.reshape(EG),
                           dst_g.reshape(EG))
    un = _wscatter(xw.reshape(NSC * NP_, HH), ex, src_g, dst_g)
    x = _gat_epilogue(un.reshape(2, NP_, HH), den.reshape(2, NP_), b_gat[i],
                      pos_split, rbf, last=(i == L_GCN - 1))

  emb, logp = _final(x, W_last, b_last)
  return emb[:N], logp[:N]


# speedup vs baseline: 15.8215x; 1.0194x over previous
import functools

import jax
import jax.numpy as jnp
from jax import lax
from jax.experimental import pallas as pl
from jax.experimental.pallas import tpu as pltpu
from jax.experimental.pallas import tpu_sc as plsc

N = 10000
E = 160000
D_IN = 256
H = 256
HH = 128
PE_DIM = 16
C = 40
L_GCN = 4
L_PE = 2
ALPHA = 0.1
NEG_SLOPE = 0.2

NP_ = 10240
EG = 180224
RG = EG // 128
BLK = 512
NSC = 2
NSUB = 16
RPN = NP_ // NSUB

def _mesh():
  return plsc.VectorSubcoreMesh(
      core_axis_name="c", subcore_axis_name="s",
      num_cores=NSC, num_subcores=NSUB)

_f32 = jnp.float32
_i32 = jnp.int32


def _mm_split(xs, Ws, b, relu_in=False):
  nx = len(xs)

  def body(*refs):
    x_refs = refs[:nx]
    w_refs = refs[nx:2 * nx]
    b_ref = refs[2 * nx]
    o_ref = refs[2 * nx + 1]
    acc = None
    for xr, wr in zip(x_refs, w_refs):
      xv = xr[...]
      if relu_in:
        xv = jnp.maximum(xv, 0.0)
      p = jnp.dot(xv, wr[...], preferred_element_type=_f32)
      acc = p if acc is None else acc + p
    acc = acc + b_ref[...]
    o_ref[0] = acc[:, :HH]
    o_ref[1] = acc[:, HH:]

  in_specs = []
  for x in xs:
    k = x.shape[1]
    in_specs.append(pl.BlockSpec((BLK, k), lambda i: (i, 0)))
  for w in Ws:
    in_specs.append(pl.BlockSpec(w.shape, lambda i: (0, 0)))
  in_specs.append(pl.BlockSpec((1, H), lambda i: (0, 0)))
  return pl.pallas_call(
      body,
      grid=(NP_ // BLK,),
      in_specs=in_specs,
      out_specs=pl.BlockSpec((2, BLK, HH), lambda i: (0, i, 0)),
      out_shape=jax.ShapeDtypeStruct((2, NP_, HH), _f32),
  )(*xs, *Ws, b.reshape(1, H))


def _gat_pre(x, W, a_src, a_dst):
  def body(x0_ref, x1_ref, w0_ref, w1_ref, as_ref, ad_ref, xw_ref, asd_ref):
    xw = (jnp.dot(x0_ref[...], w0_ref[...], preferred_element_type=_f32) +
          jnp.dot(x1_ref[...], w1_ref[...], preferred_element_type=_f32))
    xw_ref[0] = xw[:, :HH]
    xw_ref[1] = xw[:, HH:]
    asd_ref[0] = jnp.sum(xw * as_ref[...], axis=1)
    asd_ref[1] = jnp.sum(xw * ad_ref[...], axis=1)

  return pl.pallas_call(
      body,
      grid=(NP_ // BLK,),
      in_specs=[
          pl.BlockSpec((BLK, HH), lambda i: (i, 0)),
          pl.BlockSpec((BLK, HH), lambda i: (i, 0)),
          pl.BlockSpec((HH, H), lambda i: (0, 0)),
          pl.BlockSpec((HH, H), lambda i: (0, 0)),
          pl.BlockSpec((1, H), lambda i: (0, 0)),
          pl.BlockSpec((1, H), lambda i: (0, 0)),
      ],
      out_specs=[
          pl.BlockSpec((2, BLK, HH), lambda i: (0, i, 0)),
          pl.BlockSpec((2, BLK), lambda i: (0, i)),
      ],
      out_shape=[
          jax.ShapeDtypeStruct((2, NP_, HH), _f32),
          jax.ShapeDtypeStruct((2, NP_), _f32),
      ],
  )(x[0], x[1], W[:HH], W[HH:], a_src.reshape(1, H), a_dst.reshape(1, H))


def _gat_epilogue(un, den, b, pos, rbf, last):
  def body(u_ref, d_ref, b_ref, p_ref, r_ref, o_ref):
    den = d_ref[0] + d_ref[1] + 1e-16
    for h in range(2):
      xh = u_ref[h] / den[:, None] + b_ref[0, h * HH:(h + 1) * HH]
      if not last:
        rb = r_ref[0, 0]
        mix = xh * (1.0 - ALPHA) + p_ref[h] * ALPHA
        xh = jnp.maximum(rb * mix + (1.0 - rb) * xh, 0.0)
      o_ref[h] = xh

  return pl.pallas_call(
      body,
      grid=(NP_ // BLK,),
      in_specs=[
          pl.BlockSpec((2, BLK, HH), lambda i: (0, i, 0)),
          pl.BlockSpec((2, BLK), lambda i: (0, i)),
          pl.BlockSpec((1, H), lambda i: (0, 0)),
          pl.BlockSpec((2, BLK, HH), lambda i: (0, i, 0)),
          pl.BlockSpec((1, 1), lambda i: (0, 0)),
      ],
      out_specs=pl.BlockSpec((2, BLK, HH), lambda i: (0, i, 0)),
      out_shape=jax.ShapeDtypeStruct((2, NP_, HH), _f32),
  )(un, den, b.reshape(1, H), pos, rbf)


def _pe_finish(h):
  def body(h_ref, o_ref):
    o_ref[...] = jnp.tanh(jnp.maximum(h_ref[...], 0.0))

  return pl.pallas_call(
      body,
      grid=(NP_ // BLK,),
      in_specs=[pl.BlockSpec((2, BLK, HH), lambda i: (0, i, 0))],
      out_specs=pl.BlockSpec((2, BLK, HH), lambda i: (0, i, 0)),
      out_shape=jax.ShapeDtypeStruct((2, NP_, HH), _f32),
  )(h)


def _final(x, W, b):
  def body(x0_ref, x1_ref, w0_ref, w1_ref, b_ref, e_ref, l_ref):
    e = (jnp.dot(x0_ref[...], w0_ref[...], preferred_element_type=_f32) +
         jnp.dot(x1_ref[...], w1_ref[...], preferred_element_type=_f32) +
         b_ref[...])
    m = jnp.max(e, axis=1, keepdims=True)
    z = e - m
    lse = jnp.log(jnp.sum(jnp.exp(z), axis=1, keepdims=True))
    e_ref[...] = e
    l_ref[...] = z - lse

  return pl.pallas_call(
      body,
      grid=(NP_ // BLK,),
      in_specs=[
          pl.BlockSpec((BLK, HH), lambda i: (i, 0)),
          pl.BlockSpec((BLK, HH), lambda i: (i, 0)),
          pl.BlockSpec((HH, C), lambda i: (0, 0)),
          pl.BlockSpec((HH, C), lambda i: (0, 0)),
          pl.BlockSpec((1, C), lambda i: (0, 0)),
      ],
      out_specs=[
          pl.BlockSpec((BLK, C), lambda i: (i, 0)),
          pl.BlockSpec((BLK, C), lambda i: (i, 0)),
      ],
      out_shape=[
          jax.ShapeDtypeStruct((NP_, C), _f32),
          jax.ShapeDtypeStruct((NP_, C), _f32),
      ],
  )(x[0], x[1], W[:HH], W[HH:], b.reshape(1, C))


EW = EG // (NSC * NSUB)


def _build_edge_logits():
  return functools.partial(
      pl.kernel,
      out_type=(
          jax.ShapeDtypeStruct((EG,), _f32),
          jax.ShapeDtypeStruct((NSC * NP_,), _f32),
      ),
      mesh=_mesh(),
      compiler_params=pltpu.CompilerParams(needs_layout_passes=False),
      scratch_types=[
          pltpu.VMEM((NP_,), _f32),
          pltpu.VMEM((NP_,), _f32),
          pltpu.VMEM((EW,), _i32),
          pltpu.VMEM((EW,), _i32),
          pltpu.VMEM((EW,), _f32),
          pltpu.VMEM((NP_,), _f32),
          pltpu.VMEM((RPN,), _f32),
          pltpu.VMEM((RPN,), _f32),
          pltpu.VMEM_SHARED((NSUB, NP_), _f32),
      ],
  )(_edge_logits_body)


def _edge_logits_body(asd_hbm, src_hbm, dst_hbm, ex_hbm, den_hbm,
                      as_v, ad_v, src_v, dst_v, ex_v, dp_v, acc_v, red_v,
                      shared):
  c = lax.axis_index("c")
  s = lax.axis_index("s")
  wid = s * NSC + c
  pltpu.sync_copy(asd_hbm.at[pl.ds(0, NP_)], as_v)
  pltpu.sync_copy(asd_hbm.at[pl.ds(NP_, NP_)], ad_v)
  pltpu.sync_copy(src_hbm.at[pl.ds(wid * EW, EW)], src_v)
  pltpu.sync_copy(dst_hbm.at[pl.ds(wid * EW, EW)], dst_v)

  @plsc.parallel_loop(0, NP_ // 16, step=1, unroll=8)
  def _(i):
    dp_v[pl.ds(i * 16, 16)] = jnp.zeros((16,), _f32)

  @plsc.parallel_loop(0, EW // 16, step=1, unroll=8)
  def _(g):
    sl = pl.ds(g * 16, 16)
    si = src_v[sl]
    di = dst_v[sl]
    ev = plsc.load_gather(as_v, [si]) + plsc.load_gather(ad_v, [di])
    ev = jnp.where(ev >= 0.0, ev, ev * NEG_SLOPE)
    ex = jnp.exp(ev)
    ex_v[sl] = ex
    plsc.addupdate_scatter(dp_v, [di], ex)

  pltpu.sync_copy(ex_v, ex_hbm.at[pl.ds(wid * EW, EW)])
  pltpu.sync_copy(dp_v, shared.at[s])
  plsc.subcore_barrier()

  @plsc.parallel_loop(0, RPN // 16, step=1, unroll=8)
  def _(i):
    acc_v[pl.ds(i * 16, 16)] = jnp.zeros((16,), _f32)
  for i in range(NSUB):
    pltpu.sync_copy(shared.at[i, pl.ds(s * RPN, RPN)], red_v)

    @plsc.parallel_loop(0, RPN // 16, step=1, unroll=8)
    def _(j):
      sl = pl.ds(j * 16, 16)
      acc_v[sl] = acc_v[sl] + red_v[sl]
  pltpu.sync_copy(acc_v, den_hbm.at[pl.ds(c * NP_ + s * RPN, RPN)])


CH = 8


def _make_wscatter(R):
  rpw = R // NSUB

  def wscatter(tbl_hbm, w_hbm, src_hbm, dst_hbm, out_hbm,
               src_v, dst_v, w_v, rows0, rows1, acc, sem0, sem1, semS0,
               semS1):
    c = lax.axis_index("c")
    s = lax.axis_index("s")

    @plsc.parallel_loop(0, 128, step=1, unroll=4)
    def _(r):
      for k in range(8):
        rows0[r, pl.ds(k * 16, 16)] = jnp.zeros((16,), _f32)
    for i in range(RPN // 128):
      pltpu.sync_copy(rows0, acc.at[pl.ds(s * RPN + i * 128, 128)])

    plsc.subcore_barrier()

    def chunk(ci, _):
      @pl.when(ci > 0)
      def _():
        pltpu.make_async_copy(rows1, acc.at[dst_v.at[CH - 1]], semS1).wait()
      base = s * rpw + ci * CH
      pltpu.sync_copy(src_hbm.at[pl.ds(base, CH)], src_v)
      pltpu.sync_copy(dst_hbm.at[pl.ds(base, CH)], dst_v)
      pltpu.sync_copy(w_hbm.at[pl.ds(base * 128, CH * 128)], w_v)

      @plsc.parallel_loop(0, CH, step=1, unroll=4)
      def _(r):
        for k in range(8):
          sl = pl.ds(k * 16, 16)
          src_v[r, sl] = src_v[r, sl] + c * NP_

      def scale(buf, j):
        @plsc.parallel_loop(0, 128, step=1, unroll=8)
        def _(e):
          wb = plsc.load_gather(w_v, [jnp.full((16,), j * 128 + e, _i32)])
          for k in range(8):
            sl = pl.ds(k * 16, 16)
            buf[e, sl] = buf[e, sl] * wb

      pltpu.async_copy(tbl_hbm.at[src_v.at[0]], rows0, sem0)

      def pair(jj, _):
        r0 = 2 * jj

        @pl.when(jj > 0)
        def _():
          pltpu.make_async_copy(rows1, acc.at[dst_v.at[r0]], semS1).wait()
        pltpu.async_copy(tbl_hbm.at[src_v.at[r0 + 1]], rows1, sem1)
        pltpu.make_async_copy(tbl_hbm.at[src_v.at[r0]], rows0, sem0).wait()
        scale(rows0, r0)
        sc0 = pltpu.async_copy(rows0, acc.at[dst_v.at[r0]], semS0, add=True)
        pltpu.make_async_copy(tbl_hbm.at[src_v.at[r0 + 1]], rows1, sem1).wait()
        scale(rows1, r0 + 1)
        sc0.wait()

        @pl.when(jj < CH // 2 - 1)
        def _():
          pltpu.async_copy(tbl_hbm.at[src_v.at[r0 + 2]], rows0, sem0)
        pltpu.async_copy(rows1, acc.at[dst_v.at[r0 + 1]], semS1, add=True)
        return 0
      lax.fori_loop(0, CH // 2, pair, 0)
      return 0
    lax.fori_loop(0, rpw // CH, chunk, 0)
    pltpu.make_async_copy(rows1, acc.at[dst_v.at[CH - 1]], semS1).wait()

    plsc.subcore_barrier()
    pltpu.sync_copy(acc.at[pl.ds(s * RPN, RPN)],
                    out_hbm.at[pl.ds(c * NP_ + s * RPN, RPN)])

  return functools.partial(
      pl.kernel,
      out_type=jax.ShapeDtypeStruct((NSC * NP_, HH), _f32),
      mesh=_mesh(),
      compiler_params=pltpu.CompilerParams(needs_layout_passes=False),
      scratch_types=[
          pltpu.VMEM((CH, 128), _i32),
          pltpu.VMEM((CH, 128), _i32),
          pltpu.VMEM((CH * 128,), _f32),
          pltpu.VMEM((128, HH), _f32),
          pltpu.VMEM((128, HH), _f32),
          pltpu.VMEM_SHARED((NP_, HH), _f32),
          pltpu.SemaphoreType.DMA,
          pltpu.SemaphoreType.DMA,
          pltpu.SemaphoreType.DMA,
          pltpu.SemaphoreType.DMA,
      ],
  )(wscatter)


def kernel(x_h, adj, edge_index, pos_feat, run_base, W_pos, b_pos, W_pe, b_pe,
           W_init, b_init, W_gat, att_src, att_dst, b_gat, W_last, b_last):
  ei = edge_index.astype(_i32)
  loops = jnp.arange(N, dtype=_i32)
  padg = N + jnp.arange(EG - E - N, dtype=_i32) % (NP_ - N)
  src_g = jnp.concatenate([ei[0], loops, padg]).reshape(RG, 128)
  dst_g = jnp.concatenate([ei[1], loops, padg]).reshape(RG, 128)
  pads = N + jnp.arange(EG - E, dtype=_i32) % (NP_ - N)
  src_s = jnp.concatenate([ei[0], pads]).reshape(RG, 128)
  dst_s = jnp.concatenate([ei[1], pads]).reshape(RG, 128)
  w_s = jnp.concatenate([adj, jnp.zeros((EG - E,), _f32)])

  x_h_p = jnp.pad(x_h, ((0, NP_ - N), (0, 0)))
  pos_p = jnp.pad(pos_feat, ((0, NP_ - N), (0, 0)))
  rbf = jnp.asarray(jnp.asarray(run_base) == 0, _f32).reshape(1, 1)

  _edge_logits = _build_edge_logits()
  _wscatter = _make_wscatter(RG)

  h = _mm_split([pos_p], [W_pos], b_pos)
  for i in range(L_PE):
    h = _mm_split([h[0], h[1]], [W_pe[i][:HH], W_pe[i][HH:]], b_pe[i],
                  relu_in=(i > 0))
    h = _wscatter(h.reshape(NSC * NP_, HH), w_s, src_s, dst_s)
    h = h.reshape(2, NP_, HH)
  pos_split = _pe_finish(h)

  x = _mm_split([x_h_p], [W_init], b_init)
  for i in range(L_GCN):
    xw, a_sd = _gat_pre(x, W_gat[i], att_src[i], att_dst[i])
    ex, den = _edge_logits(a_sd.reshape(NSC * NP_), src_g.reshape(EG),
                           dst_g.reshape(EG))
    un = _wscatter(xw.reshape(NSC * NP_, HH), ex, src_g, dst_g)
    x = _gat_epilogue(un.reshape(2, NP_, HH), den.reshape(2, NP_), b_gat[i],
                      pos_split, rbf, last=(i == L_GCN - 1))

  emb, logp = _final(x, W_last, b_last)
  return emb[:N], logp[:N]
